# baseline (speedup 1.0000x reference)
import jax
import jax.numpy as jnp
from jax.experimental import pallas as pl
from jax.experimental.pallas import tpu as pltpu
from jax.experimental.pallas import tpu_sc as plsc

_B, _E, _C, _O = 16, 576, 256, 8192
_N = _B * _E
_BETA = 0.25
_NOISE_STD = 0.5
_R = 512
_NR = _N // _R
_KT = 2048
_NK = _O // _KT
_GW = 128


def _csq_body(cb_ref, csq_ref):
    c = cb_ref[...]
    csq_ref[...] = jnp.sum(c * c, axis=1, keepdims=True)


def _dist_body(x_ref, wpre_ref, bpre_ref, cbt_ref, csq_ref, noise_ref,
               lat_ref, idet_ref, inoisy_ref, mind_ref,
               lat_s, lath_s, zsq_s, rmin_s, ridx_s):
    j = pl.program_id(1)

    @pl.when(j == 0)
    def _init():
        lat0 = jnp.dot(x_ref[...].astype(jnp.bfloat16), wpre_ref[...],
                       preferred_element_type=jnp.float32) + bpre_ref[...]
        lat_s[...] = lat0
        lath_s[...] = (lat0.astype(jnp.bfloat16) * jnp.bfloat16(2.0))
        lat_ref[...] = lat0
        zsq_s[...] = jnp.sum(lat0 * lat0, axis=1, keepdims=True)
        rmin_s[...] = jnp.full((_R, 1), jnp.inf, jnp.float32)
        ridx_s[...] = jnp.zeros((_R, 1), jnp.int32)

    mm2 = jnp.dot(lath_s[...], cbt_ref[...],
                  preferred_element_type=jnp.float32)
    dist = (zsq_s[...] + csq_ref[...]) - mm2
    tmin = jnp.min(dist, axis=1, keepdims=True)
    gidx = jax.lax.broadcasted_iota(jnp.int32, (_R, _KT), 1)
    tidx = (jnp.min(jnp.where(dist == tmin, gidx, jnp.int32(2 ** 30)),
                    axis=1, keepdims=True) + j * _KT)
    upd = tmin < rmin_s[...]
    rmin_s[...] = jnp.where(upd, tmin, rmin_s[...])
    ridx_s[...] = jnp.where(upd, tidx, ridx_s[...])

    @pl.when(j == _NK - 1)
    def _fin():
        idet = ridx_s[...]
        idet_ref[...] = idet
        mind_ref[...] = rmin_s[...]
        off = jnp.round(noise_ref[...] * _NOISE_STD).astype(jnp.int32)
        inoisy_ref[...] = jnp.clip(idet + off, 0, _O - 1)


def _sc_gather(codebook, idx_flat):
    n = idx_flat.shape[1]
    mesh = plsc.VectorSubcoreMesh(core_axis_name="core",
                                  subcore_axis_name="subcore")

    @pl.kernel(out_type=jax.ShapeDtypeStruct((n, _C), jnp.float32),
               mesh=mesh)
    def kern(cb_hbm, i_hbm, o_hbm):
        def body(i_vmem, o_vmem):
            pltpu.sync_copy(cb_hbm.at[i_vmem.at[0]], o_vmem)

        pltpu.emit_pipeline(
            body,
            grid=(n // _GW,),
            in_specs=[pl.BlockSpec((1, _GW), index_map=lambda i: (0, i))],
            out_specs=[pl.BlockSpec((_GW, _C), index_map=lambda i: (i, 0))],
            core_axis_name=("core", "subcore"),
            dimension_semantics=(pltpu.PARALLEL,),
        )(i_hbm, o_hbm)

    return kern(codebook, idx_flat)


def _out_body(lat_ref, q_ref, y_ref, wpost_ref, bpost_ref, mind_ref,
              out_ref, rp_ref, cp_ref, ep_ref):
    lat = lat_ref[...]
    q = q_ref[...]
    st = lat + (q - lat)
    o = jnp.dot(st.astype(jnp.bfloat16), wpost_ref[...],
                preferred_element_type=jnp.float32) + bpost_ref[...]
    out_ref[...] = o
    d0 = o - y_ref[...]
    d2 = q - lat
    z128 = jnp.zeros((1, 1, 128), jnp.float32)
    rsum = jnp.sum(jnp.sum(d0 * d0, axis=1, keepdims=True), axis=0,
                   keepdims=True)
    esum = jnp.sum(jnp.sum(d2 * d2, axis=1, keepdims=True), axis=0,
                   keepdims=True)
    csum = jnp.sum(mind_ref[...], axis=0, keepdims=True)
    rp_ref[...] = z128 + rsum.reshape(1, 1, 1)
    ep_ref[...] = z128 + esum.reshape(1, 1, 1)
    cp_ref[...] = z128 + csum.reshape(1, 1, 1)


def _loss_body(rp_ref, cp_ref, ep_ref, loss_ref):
    ntot = float(_N * _C)
    recon = jnp.sum(rp_ref[...], axis=0, keepdims=True) / ntot
    commit = _BETA * (jnp.sum(cp_ref[...], axis=0, keepdims=True) / ntot)
    embed = jnp.sum(ep_ref[...], axis=0, keepdims=True) / ntot
    loss_ref[...] = (recon + commit) + embed


def _dist_call(xf, noise, wpre_bf, bpre, cbt_bf, csq):
    n = xf.shape[0]
    nr = n // _R
    return pl.pallas_call(
        _dist_body,
        grid=(nr, _NK),
        in_specs=[
            pl.BlockSpec((_R, _C), lambda i, j: (i, 0)),
            pl.BlockSpec((_C, _C), lambda i, j: (0, 0)),
            pl.BlockSpec((1, _C), lambda i, j: (0, 0)),
            pl.BlockSpec((_C, _KT), lambda i, j: (0, j)),
            pl.BlockSpec((1, _KT), lambda i, j: (0, j)),
            pl.BlockSpec((_R, 1), lambda i, j: (i, 0)),
        ],
        out_specs=[
            pl.BlockSpec((_R, _C), lambda i, j: (i, 0)),
            pl.BlockSpec((_R, 1), lambda i, j: (i, 0)),
            pl.BlockSpec((_R, 1), lambda i, j: (i, 0)),
            pl.BlockSpec((_R, 1), lambda i, j: (i, 0)),
        ],
        out_shape=[
            jax.ShapeDtypeStruct((n, _C), jnp.float32),
            jax.ShapeDtypeStruct((n, 1), jnp.int32),
            jax.ShapeDtypeStruct((n, 1), jnp.int32),
            jax.ShapeDtypeStruct((n, 1), jnp.float32),
        ],
        scratch_shapes=[
            pltpu.VMEM((_R, _C), jnp.float32),
            pltpu.VMEM((_R, _C), jnp.bfloat16),
            pltpu.VMEM((_R, 1), jnp.float32),
            pltpu.VMEM((_R, 1), jnp.float32),
            pltpu.VMEM((_R, 1), jnp.int32),
        ],
        compiler_params=pltpu.CompilerParams(
            dimension_semantics=("parallel", "arbitrary")),
    )(xf, wpre_bf, bpre, cbt_bf, csq, noise)


def _out_call(lat, q_noisy, yf, wpost_bf, bpost, mind):
    n = lat.shape[0]
    nr = n // _R
    return pl.pallas_call(
        _out_body,
        grid=(nr,),
        in_specs=[
            pl.BlockSpec((_R, _C), lambda i: (i, 0)),
            pl.BlockSpec((_R, _C), lambda i: (i, 0)),
            pl.BlockSpec((_R, _C), lambda i: (i, 0)),
            pl.BlockSpec((_C, _C), lambda i: (0, 0)),
            pl.BlockSpec((1, _C), lambda i: (0, 0)),
            pl.BlockSpec((_R, 1), lambda i: (i, 0)),
        ],
        out_specs=[
            pl.BlockSpec((_R, _C), lambda i: (i, 0)),
            pl.BlockSpec((1, 1, 128), lambda i: (i, 0, 0)),
            pl.BlockSpec((1, 1, 128), lambda i: (i, 0, 0)),
            pl.BlockSpec((1, 1, 128), lambda i: (i, 0, 0)),
        ],
        out_shape=[
            jax.ShapeDtypeStruct((n, _C), jnp.float32),
            jax.ShapeDtypeStruct((nr, 1, 128), jnp.float32),
            jax.ShapeDtypeStruct((nr, 1, 128), jnp.float32),
            jax.ShapeDtypeStruct((nr, 1, 128), jnp.float32),
        ],
        compiler_params=pltpu.CompilerParams(
            dimension_semantics=("parallel",)),
    )(lat, q_noisy, yf, wpost_bf, bpost, mind)


_NCHUNK = 1


def kernel(x, y, noise, W_pre, b_pre, W_post, b_post, codebook):
    xf = x.reshape(_N, _C)
    yf = y.reshape(_N, _C)
    wpre_bf = W_pre.astype(jnp.bfloat16)
    wpost_bf = W_post.astype(jnp.bfloat16)
    bpre = b_pre.reshape(1, _C)
    bpost = b_post.reshape(1, _C)
    cbt_bf = codebook.T.astype(jnp.bfloat16)

    csq_col = pl.pallas_call(
        _csq_body,
        out_shape=jax.ShapeDtypeStruct((_O, 1), jnp.float32),
    )(codebook)
    csq = csq_col.reshape(1, _O)

    nc = _N // _NCHUNK
    dist_res = []
    for c in range(_NCHUNK):
        s = slice(c * nc, (c + 1) * nc)
        dist_res.append(_dist_call(xf[s], noise[s], wpre_bf, bpre,
                                   cbt_bf, csq))
    gathers = [_sc_gather(codebook, dist_res[c][2].reshape(1, nc))
               for c in range(_NCHUNK)]
    outs = []
    for c in range(_NCHUNK):
        lat, _idet, _inoisy, mind = dist_res[c]
        s = slice(c * nc, (c + 1) * nc)
        outs.append(_out_call(lat, gathers[c], yf[s], wpost_bf, bpost, mind))

    out_f = jnp.concatenate([o[0] for o in outs], axis=0)
    rp = jnp.concatenate([o[1] for o in outs], axis=0)
    cp = jnp.concatenate([o[2] for o in outs], axis=0)
    ep = jnp.concatenate([o[3] for o in outs], axis=0)

    lossv = pl.pallas_call(
        _loss_body,
        out_shape=jax.ShapeDtypeStruct((1, 1, 128), jnp.float32),
    )(rp, cp, ep)

    return out_f.reshape(_B, _E, _C), lossv[0, 0, 0]


# speedup vs baseline: 1.0299x; 1.0299x over previous
import jax
import jax.numpy as jnp
from jax.experimental import pallas as pl
from jax.experimental.pallas import tpu as pltpu
from jax.experimental.pallas import tpu_sc as plsc

_B, _E, _C, _O = 16, 576, 256, 8192
_N = _B * _E
_BETA = 0.25
_NOISE_STD = 0.5
_R = 512
_NR = _N // _R
_KT = 2048
_NK = _O // _KT
_GW = 128


def _csq_body(cb_ref, csq_ref):
    c = cb_ref[...]
    csq_ref[...] = jnp.sum(c * c, axis=1, keepdims=True)


def _dist_body(x_ref, wpre_ref, bpre_ref, cbt_ref, csq_ref, noise_ref,
               lat_ref, idet_ref, inoisy_ref, mind_ref,
               lat_s, lath_s, zsq_s, rmin_s, ridx_s):
    j = pl.program_id(1)

    @pl.when(j == 0)
    def _init():
        lat0 = jnp.dot(x_ref[...].astype(jnp.bfloat16), wpre_ref[...],
                       preferred_element_type=jnp.float32) + bpre_ref[...]
        lat_s[...] = lat0
        lath_s[...] = lat0.astype(jnp.bfloat16)
        lat_ref[...] = lat0
        zsq_s[...] = jnp.sum(lat0 * lat0, axis=1, keepdims=True)
        rmin_s[...] = jnp.full((_R, 1), jnp.inf, jnp.float32)
        ridx_s[...] = jnp.zeros((_R, 1), jnp.int32)

    mm = jnp.dot(lath_s[...], cbt_ref[...],
                 preferred_element_type=jnp.float32)
    dist = (zsq_s[...] + csq_ref[...]) - 2.0 * mm
    tmin = jnp.min(dist, axis=1, keepdims=True)
    gidx = jax.lax.broadcasted_iota(jnp.int32, (_R, _KT), 1)
    tidx = (jnp.min(jnp.where(dist == tmin, gidx, jnp.int32(2 ** 30)),
                    axis=1, keepdims=True) + j * _KT)
    upd = tmin < rmin_s[...]
    rmin_s[...] = jnp.where(upd, tmin, rmin_s[...])
    ridx_s[...] = jnp.where(upd, tidx, ridx_s[...])

    @pl.when(j == _NK - 1)
    def _fin():
        idet = ridx_s[...]
        idet_ref[...] = idet
        mind_ref[...] = rmin_s[...]
        off = jnp.round(noise_ref[...] * _NOISE_STD).astype(jnp.int32)
        inoisy_ref[...] = jnp.clip(idet + off, 0, _O - 1)


def _sc_gather(codebook, idx_flat):
    n = idx_flat.shape[1]
    mesh = plsc.VectorSubcoreMesh(core_axis_name="core",
                                  subcore_axis_name="subcore")

    @pl.kernel(out_type=jax.ShapeDtypeStruct((n, _C), jnp.float32),
               mesh=mesh)
    def kern(cb_hbm, i_hbm, o_hbm):
        def body(i_vmem, o_vmem):
            pltpu.sync_copy(cb_hbm.at[i_vmem.at[0]], o_vmem)

        pltpu.emit_pipeline(
            body,
            grid=(n // _GW,),
            in_specs=[pl.BlockSpec((1, _GW), index_map=lambda i: (0, i))],
            out_specs=[pl.BlockSpec((_GW, _C), index_map=lambda i: (i, 0))],
            core_axis_name=("core", "subcore"),
            dimension_semantics=(pltpu.PARALLEL,),
        )(i_hbm, o_hbm)

    return kern(codebook, idx_flat)


def _out_body(lat_ref, q_ref, y_ref, wpost_ref, bpost_ref, mind_ref,
              out_ref, rp_ref, cp_ref, ep_ref):
    lat = lat_ref[...]
    q = q_ref[...]
    st = lat + (q - lat)
    o = jnp.dot(st.astype(jnp.bfloat16), wpost_ref[...],
                preferred_element_type=jnp.float32) + bpost_ref[...]
    out_ref[...] = o
    d0 = o - y_ref[...]
    d2 = q - lat
    z128 = jnp.zeros((1, 1, 128), jnp.float32)
    rsum = jnp.sum(jnp.sum(d0 * d0, axis=1, keepdims=True), axis=0,
                   keepdims=True)
    esum = jnp.sum(jnp.sum(d2 * d2, axis=1, keepdims=True), axis=0,
                   keepdims=True)
    csum = jnp.sum(mind_ref[...], axis=0, keepdims=True)
    rp_ref[...] = z128 + rsum.reshape(1, 1, 1)
    ep_ref[...] = z128 + esum.reshape(1, 1, 1)
    cp_ref[...] = z128 + csum.reshape(1, 1, 1)


def _loss_body(rp_ref, cp_ref, ep_ref, loss_ref):
    ntot = float(_N * _C)
    recon = jnp.sum(rp_ref[...], axis=0, keepdims=True) / ntot
    commit = _BETA * (jnp.sum(cp_ref[...], axis=0, keepdims=True) / ntot)
    embed = jnp.sum(ep_ref[...], axis=0, keepdims=True) / ntot
    loss_ref[...] = (recon + commit) + embed


def _dist_call(xf, noise, wpre_bf, bpre, cbt_bf, csq):
    n = xf.shape[0]
    nr = n // _R
    return pl.pallas_call(
        _dist_body,
        grid=(nr, _NK),
        in_specs=[
            pl.BlockSpec((_R, _C), lambda i, j: (i, 0)),
            pl.BlockSpec((_C, _C), lambda i, j: (0, 0)),
            pl.BlockSpec((1, _C), lambda i, j: (0, 0)),
            pl.BlockSpec((_C, _KT), lambda i, j: (0, j)),
            pl.BlockSpec((1, _KT), lambda i, j: (0, j)),
            pl.BlockSpec((_R, 1), lambda i, j: (i, 0)),
        ],
        out_specs=[
            pl.BlockSpec((_R, _C), lambda i, j: (i, 0)),
            pl.BlockSpec((_R, 1), lambda i, j: (i, 0)),
            pl.BlockSpec((_R, 1), lambda i, j: (i, 0)),
            pl.BlockSpec((_R, 1), lambda i, j: (i, 0)),
        ],
        out_shape=[
            jax.ShapeDtypeStruct((n, _C), jnp.float32),
            jax.ShapeDtypeStruct((n, 1), jnp.int32),
            jax.ShapeDtypeStruct((n, 1), jnp.int32),
            jax.ShapeDtypeStruct((n, 1), jnp.float32),
        ],
        scratch_shapes=[
            pltpu.VMEM((_R, _C), jnp.float32),
            pltpu.VMEM((_R, _C), jnp.bfloat16),
            pltpu.VMEM((_R, 1), jnp.float32),
            pltpu.VMEM((_R, 1), jnp.float32),
            pltpu.VMEM((_R, 1), jnp.int32),
        ],
        compiler_params=pltpu.CompilerParams(
            dimension_semantics=("parallel", "arbitrary")),
    )(xf, wpre_bf, bpre, cbt_bf, csq, noise)


def _out_call(lat, q_noisy, yf, wpost_bf, bpost, mind):
    n = lat.shape[0]
    nr = n // _R
    return pl.pallas_call(
        _out_body,
        grid=(nr,),
        in_specs=[
            pl.BlockSpec((_R, _C), lambda i: (i, 0)),
            pl.BlockSpec((_R, _C), lambda i: (i, 0)),
            pl.BlockSpec((_R, _C), lambda i: (i, 0)),
            pl.BlockSpec((_C, _C), lambda i: (0, 0)),
            pl.BlockSpec((1, _C), lambda i: (0, 0)),
            pl.BlockSpec((_R, 1), lambda i: (i, 0)),
        ],
        out_specs=[
            pl.BlockSpec((_R, _C), lambda i: (i, 0)),
            pl.BlockSpec((1, 1, 128), lambda i: (i, 0, 0)),
            pl.BlockSpec((1, 1, 128), lambda i: (i, 0, 0)),
            pl.BlockSpec((1, 1, 128), lambda i: (i, 0, 0)),
        ],
        out_shape=[
            jax.ShapeDtypeStruct((n, _C), jnp.float32),
            jax.ShapeDtypeStruct((nr, 1, 128), jnp.float32),
            jax.ShapeDtypeStruct((nr, 1, 128), jnp.float32),
            jax.ShapeDtypeStruct((nr, 1, 128), jnp.float32),
        ],
        compiler_params=pltpu.CompilerParams(
            dimension_semantics=("parallel",)),
    )(lat, q_noisy, yf, wpost_bf, bpost, mind)


_NCHUNK = 2


def kernel(x, y, noise, W_pre, b_pre, W_post, b_post, codebook):
    xf = x.reshape(_N, _C)
    yf = y.reshape(_N, _C)
    wpre_bf = W_pre.astype(jnp.bfloat16)
    wpost_bf = W_post.astype(jnp.bfloat16)
    bpre = b_pre.reshape(1, _C)
    bpost = b_post.reshape(1, _C)
    cbt_bf = codebook.T.astype(jnp.bfloat16)

    csq_col = pl.pallas_call(
        _csq_body,
        out_shape=jax.ShapeDtypeStruct((_O, 1), jnp.float32),
    )(codebook)
    csq = csq_col.reshape(1, _O)

    nc = _N // _NCHUNK
    dist_res = []
    gathers = []
    for c in range(_NCHUNK):
        s = slice(c * nc, (c + 1) * nc)
        dist_res.append(_dist_call(xf[s], noise[s], wpre_bf, bpre,
                                   cbt_bf, csq))
        gathers.append(_sc_gather(codebook, dist_res[c][2].reshape(1, nc)))
    outs = []
    for c in range(_NCHUNK):
        lat, _idet, _inoisy, mind = dist_res[c]
        s = slice(c * nc, (c + 1) * nc)
        outs.append(_out_call(lat, gathers[c], yf[s], wpost_bf, bpost, mind))

    out_f = jnp.concatenate([o[0] for o in outs], axis=0)
    rp = jnp.concatenate([o[1] for o in outs], axis=0)
    cp = jnp.concatenate([o[2] for o in outs], axis=0)
    ep = jnp.concatenate([o[3] for o in outs], axis=0)

    lossv = pl.pallas_call(
        _loss_body,
        out_shape=jax.ShapeDtypeStruct((1, 1, 128), jnp.float32),
    )(rp, cp, ep)

    return out_f.reshape(_B, _E, _C), lossv[0, 0, 0]


# speedup vs baseline: 1.1833x; 1.1490x over previous
import jax
import jax.numpy as jnp
from jax.experimental import pallas as pl
from jax.experimental.pallas import tpu as pltpu
from jax.experimental.pallas import tpu_sc as plsc

_B, _E, _C, _O = 16, 576, 256, 8192
_N = _B * _E
_BETA = 0.25
_NOISE_STD = 0.5
_R = 512
_NR = _N // _R
_KT = 8192
_NK = _O // _KT
_GW = 128


def _csq_body(cb_ref, csq_ref):
    c = cb_ref[...]
    csq_ref[...] = jnp.sum(c * c, axis=1, keepdims=True)


def _dist_body(x_ref, wpre_ref, bpre_ref, cbt_ref, csq_ref, noise_ref,
               lat_ref, idet_ref, inoisy_ref, mind_ref,
               lat_s, lath_s, zsq_s, rmin_s, ridx_s):
    j = pl.program_id(1)

    @pl.when(j == 0)
    def _init():
        lat0 = jnp.dot(x_ref[...].astype(jnp.bfloat16), wpre_ref[...],
                       preferred_element_type=jnp.float32) + bpre_ref[...]
        lat_s[...] = lat0
        lath_s[...] = lat0.astype(jnp.bfloat16)
        lat_ref[...] = lat0
        zsq_s[...] = jnp.sum(lat0 * lat0, axis=1, keepdims=True)
        rmin_s[...] = jnp.full((_R, 1), jnp.inf, jnp.float32)
        ridx_s[...] = jnp.zeros((_R, 1), jnp.int32)

    mm = jnp.dot(lath_s[...], cbt_ref[...],
                 preferred_element_type=jnp.float32)
    dist = (zsq_s[...] + csq_ref[...]) - 2.0 * mm
    tmin = jnp.min(dist, axis=1, keepdims=True)
    gidx = jax.lax.broadcasted_iota(jnp.int32, (_R, _KT), 1)
    tidx = (jnp.min(jnp.where(dist == tmin, gidx, jnp.int32(2 ** 30)),
                    axis=1, keepdims=True) + j * _KT)
    upd = tmin < rmin_s[...]
    rmin_s[...] = jnp.where(upd, tmin, rmin_s[...])
    ridx_s[...] = jnp.where(upd, tidx, ridx_s[...])

    @pl.when(j == _NK - 1)
    def _fin():
        idet = ridx_s[...]
        idet_ref[...] = idet
        mind_ref[...] = rmin_s[...]
        off = jnp.round(noise_ref[...] * _NOISE_STD).astype(jnp.int32)
        inoisy_ref[...] = jnp.clip(idet + off, 0, _O - 1)


def _sc_gather(codebook, idx_flat):
    n = idx_flat.shape[1]
    mesh = plsc.VectorSubcoreMesh(core_axis_name="core",
                                  subcore_axis_name="subcore")

    @pl.kernel(out_type=jax.ShapeDtypeStruct((n, _C), jnp.float32),
               mesh=mesh)
    def kern(cb_hbm, i_hbm, o_hbm):
        def body(i_vmem, o_vmem):
            pltpu.sync_copy(cb_hbm.at[i_vmem.at[0]], o_vmem)

        pltpu.emit_pipeline(
            body,
            grid=(n // _GW,),
            in_specs=[pl.BlockSpec((1, _GW), index_map=lambda i: (0, i))],
            out_specs=[pl.BlockSpec((_GW, _C), index_map=lambda i: (i, 0))],
            core_axis_name=("core", "subcore"),
            dimension_semantics=(pltpu.PARALLEL,),
        )(i_hbm, o_hbm)

    return kern(codebook, idx_flat)


def _out_body(lat_ref, q_ref, y_ref, wpost_ref, bpost_ref, mind_ref,
              out_ref, rp_ref, cp_ref, ep_ref):
    lat = lat_ref[...]
    q = q_ref[...]
    st = lat + (q - lat)
    o = jnp.dot(st.astype(jnp.bfloat16), wpost_ref[...],
                preferred_element_type=jnp.float32) + bpost_ref[...]
    out_ref[...] = o
    d0 = o - y_ref[...]
    d2 = q - lat
    z128 = jnp.zeros((1, 1, 128), jnp.float32)
    rsum = jnp.sum(jnp.sum(d0 * d0, axis=1, keepdims=True), axis=0,
                   keepdims=True)
    esum = jnp.sum(jnp.sum(d2 * d2, axis=1, keepdims=True), axis=0,
                   keepdims=True)
    csum = jnp.sum(mind_ref[...], axis=0, keepdims=True)
    rp_ref[...] = z128 + rsum.reshape(1, 1, 1)
    ep_ref[...] = z128 + esum.reshape(1, 1, 1)
    cp_ref[...] = z128 + csum.reshape(1, 1, 1)


def _loss_body(rp_ref, cp_ref, ep_ref, loss_ref):
    ntot = float(_N * _C)
    recon = jnp.sum(rp_ref[...], axis=0, keepdims=True) / ntot
    commit = _BETA * (jnp.sum(cp_ref[...], axis=0, keepdims=True) / ntot)
    embed = jnp.sum(ep_ref[...], axis=0, keepdims=True) / ntot
    loss_ref[...] = (recon + commit) + embed


def _dist_call(xf, noise, wpre_bf, bpre, cbt_bf, csq):
    n = xf.shape[0]
    nr = n // _R
    return pl.pallas_call(
        _dist_body,
        grid=(nr, _NK),
        in_specs=[
            pl.BlockSpec((_R, _C), lambda i, j: (i, 0)),
            pl.BlockSpec((_C, _C), lambda i, j: (0, 0)),
            pl.BlockSpec((1, _C), lambda i, j: (0, 0)),
            pl.BlockSpec((_C, _KT), lambda i, j: (0, j)),
            pl.BlockSpec((1, _KT), lambda i, j: (0, j)),
            pl.BlockSpec((_R, 1), lambda i, j: (i, 0)),
        ],
        out_specs=[
            pl.BlockSpec((_R, _C), lambda i, j: (i, 0)),
            pl.BlockSpec((_R, 1), lambda i, j: (i, 0)),
            pl.BlockSpec((_R, 1), lambda i, j: (i, 0)),
            pl.BlockSpec((_R, 1), lambda i, j: (i, 0)),
        ],
        out_shape=[
            jax.ShapeDtypeStruct((n, _C), jnp.float32),
            jax.ShapeDtypeStruct((n, 1), jnp.int32),
            jax.ShapeDtypeStruct((n, 1), jnp.int32),
            jax.ShapeDtypeStruct((n, 1), jnp.float32),
        ],
        scratch_shapes=[
            pltpu.VMEM((_R, _C), jnp.float32),
            pltpu.VMEM((_R, _C), jnp.bfloat16),
            pltpu.VMEM((_R, 1), jnp.float32),
            pltpu.VMEM((_R, 1), jnp.float32),
            pltpu.VMEM((_R, 1), jnp.int32),
        ],
        compiler_params=pltpu.CompilerParams(
            dimension_semantics=("parallel", "arbitrary")),
    )(xf, wpre_bf, bpre, cbt_bf, csq, noise)


def _out_call(lat, q_noisy, yf, wpost_bf, bpost, mind):
    n = lat.shape[0]
    nr = n // _R
    return pl.pallas_call(
        _out_body,
        grid=(nr,),
        in_specs=[
            pl.BlockSpec((_R, _C), lambda i: (i, 0)),
            pl.BlockSpec((_R, _C), lambda i: (i, 0)),
            pl.BlockSpec((_R, _C), lambda i: (i, 0)),
            pl.BlockSpec((_C, _C), lambda i: (0, 0)),
            pl.BlockSpec((1, _C), lambda i: (0, 0)),
            pl.BlockSpec((_R, 1), lambda i: (i, 0)),
        ],
        out_specs=[
            pl.BlockSpec((_R, _C), lambda i: (i, 0)),
            pl.BlockSpec((1, 1, 128), lambda i: (i, 0, 0)),
            pl.BlockSpec((1, 1, 128), lambda i: (i, 0, 0)),
            pl.BlockSpec((1, 1, 128), lambda i: (i, 0, 0)),
        ],
        out_shape=[
            jax.ShapeDtypeStruct((n, _C), jnp.float32),
            jax.ShapeDtypeStruct((nr, 1, 128), jnp.float32),
            jax.ShapeDtypeStruct((nr, 1, 128), jnp.float32),
            jax.ShapeDtypeStruct((nr, 1, 128), jnp.float32),
        ],
        compiler_params=pltpu.CompilerParams(
            dimension_semantics=("parallel",)),
    )(lat, q_noisy, yf, wpost_bf, bpost, mind)


_NCHUNK = 1


def kernel(x, y, noise, W_pre, b_pre, W_post, b_post, codebook):
    xf = x.reshape(_N, _C)
    yf = y.reshape(_N, _C)
    wpre_bf = W_pre.astype(jnp.bfloat16)
    wpost_bf = W_post.astype(jnp.bfloat16)
    bpre = b_pre.reshape(1, _C)
    bpost = b_post.reshape(1, _C)
    cbt_bf = codebook.T.astype(jnp.bfloat16)

    csq_col = pl.pallas_call(
        _csq_body,
        out_shape=jax.ShapeDtypeStruct((_O, 1), jnp.float32),
    )(codebook)
    csq = csq_col.reshape(1, _O)

    nc = _N // _NCHUNK
    dist_res = []
    gathers = []
    for c in range(_NCHUNK):
        s = slice(c * nc, (c + 1) * nc)
        dist_res.append(_dist_call(xf[s], noise[s], wpre_bf, bpre,
                                   cbt_bf, csq))
        gathers.append(_sc_gather(codebook, dist_res[c][2].reshape(1, nc)))
    outs = []
    for c in range(_NCHUNK):
        lat, _idet, _inoisy, mind = dist_res[c]
        s = slice(c * nc, (c + 1) * nc)
        outs.append(_out_call(lat, gathers[c], yf[s], wpost_bf, bpost, mind))

    out_f = jnp.concatenate([o[0] for o in outs], axis=0)
    rp = jnp.concatenate([o[1] for o in outs], axis=0)
    cp = jnp.concatenate([o[2] for o in outs], axis=0)
    ep = jnp.concatenate([o[3] for o in outs], axis=0)

    lossv = pl.pallas_call(
        _loss_body,
        out_shape=jax.ShapeDtypeStruct((1, 1, 128), jnp.float32),
    )(rp, cp, ep)

    return out_f.reshape(_B, _E, _C), lossv[0, 0, 0]


# speedup vs baseline: 1.2116x; 1.0239x over previous
import jax
import jax.numpy as jnp
from jax.experimental import pallas as pl
from jax.experimental.pallas import tpu as pltpu
from jax.experimental.pallas import tpu_sc as plsc

_B, _E, _C, _O = 16, 576, 256, 8192
_N = _B * _E
_BETA = 0.25
_NOISE_STD = 0.5
_R = 512
_NR = _N // _R
_KT = 8192
_NK = _O // _KT
_GW = 128


def _csq_body(cb_ref, csq_ref):
    c = cb_ref[...]
    csq_ref[...] = jnp.sum(c * c, axis=1, keepdims=True)


def _dist_body(x_ref, wpre_ref, bpre_ref, cbt_ref, csq_ref, noise_ref,
               lat_ref, idet_ref, inoisy_ref, mind_ref,
               lat_s, lath_s, zsq_s, rmin_s, ridx_s):
    j = pl.program_id(1)

    @pl.when(j == 0)
    def _init():
        lat0 = jnp.dot(x_ref[...].astype(jnp.bfloat16), wpre_ref[...],
                       preferred_element_type=jnp.float32) + bpre_ref[...]
        lat_s[...] = lat0
        lath_s[...] = lat0.astype(jnp.bfloat16)
        lat_ref[...] = lat0
        zsq_s[...] = jnp.sum(lat0 * lat0, axis=1, keepdims=True)
        rmin_s[...] = jnp.full((_R, 1), jnp.inf, jnp.float32)
        ridx_s[...] = jnp.zeros((_R, 1), jnp.int32)

    mm = jnp.dot(lath_s[...], cbt_ref[...],
                 preferred_element_type=jnp.float32)
    dist = (zsq_s[...] + csq_ref[...]) - 2.0 * mm
    tmin = jnp.min(dist, axis=1, keepdims=True)
    gidx = jax.lax.broadcasted_iota(jnp.int32, (_R, _KT), 1)
    tidx = (jnp.min(jnp.where(dist == tmin, gidx, jnp.int32(2 ** 30)),
                    axis=1, keepdims=True) + j * _KT)
    upd = tmin < rmin_s[...]
    rmin_s[...] = jnp.where(upd, tmin, rmin_s[...])
    ridx_s[...] = jnp.where(upd, tidx, ridx_s[...])

    @pl.when(j == _NK - 1)
    def _fin():
        idet = ridx_s[...]
        idet_ref[...] = idet
        mind_ref[...] = rmin_s[...]
        off = jnp.round(noise_ref[...] * _NOISE_STD).astype(jnp.int32)
        inoisy_ref[...] = jnp.clip(idet + off, 0, _O - 1)


def _sc_gather(codebook, idx_flat):
    n = idx_flat.shape[1]
    per = n // 32
    idx3 = idx_flat.reshape(32, 1, per)
    mesh = plsc.VectorSubcoreMesh(core_axis_name="core",
                                  subcore_axis_name="subcore")

    @pl.kernel(out_type=jax.ShapeDtypeStruct((n, _C), jnp.float32),
               mesh=mesh,
               scratch_types=[pltpu.VMEM((1, per), jnp.int32),
                              pltpu.VMEM((per, _C), jnp.float32)])
    def kern(cb_hbm, i_hbm, o_hbm, ivmem, ovmem):
        c = jax.lax.axis_index("core")
        s = jax.lax.axis_index("subcore")
        u = c * 16 + s
        pltpu.sync_copy(i_hbm.at[u], ivmem)
        pltpu.sync_copy(cb_hbm.at[ivmem.at[0]], ovmem)
        pltpu.sync_copy(ovmem, o_hbm.at[pl.ds(u * per, per)])

    return kern(codebook, idx3)


def _out_body(lat_ref, q_ref, y_ref, wpost_ref, bpost_ref, mind_ref,
              out_ref, rp_ref, cp_ref, ep_ref):
    lat = lat_ref[...]
    q = q_ref[...]
    st = lat + (q - lat)
    o = jnp.dot(st.astype(jnp.bfloat16), wpost_ref[...],
                preferred_element_type=jnp.float32) + bpost_ref[...]
    out_ref[...] = o
    d0 = o - y_ref[...]
    d2 = q - lat
    z128 = jnp.zeros((1, 1, 128), jnp.float32)
    rsum = jnp.sum(jnp.sum(d0 * d0, axis=1, keepdims=True), axis=0,
                   keepdims=True)
    esum = jnp.sum(jnp.sum(d2 * d2, axis=1, keepdims=True), axis=0,
                   keepdims=True)
    csum = jnp.sum(mind_ref[...], axis=0, keepdims=True)
    rp_ref[...] = z128 + rsum.reshape(1, 1, 1)
    ep_ref[...] = z128 + esum.reshape(1, 1, 1)
    cp_ref[...] = z128 + csum.reshape(1, 1, 1)


def _loss_body(rp_ref, cp_ref, ep_ref, loss_ref):
    ntot = float(_N * _C)
    recon = jnp.sum(rp_ref[...], axis=0, keepdims=True) / ntot
    commit = _BETA * (jnp.sum(cp_ref[...], axis=0, keepdims=True) / ntot)
    embed = jnp.sum(ep_ref[...], axis=0, keepdims=True) / ntot
    loss_ref[...] = (recon + commit) + embed


def _dist_call(xf, noise, wpre_bf, bpre, cbt_bf, csq):
    n = xf.shape[0]
    nr = n // _R
    return pl.pallas_call(
        _dist_body,
        grid=(nr, _NK),
        in_specs=[
            pl.BlockSpec((_R, _C), lambda i, j: (i, 0)),
            pl.BlockSpec((_C, _C), lambda i, j: (0, 0)),
            pl.BlockSpec((1, _C), lambda i, j: (0, 0)),
            pl.BlockSpec((_C, _KT), lambda i, j: (0, j)),
            pl.BlockSpec((1, _KT), lambda i, j: (0, j)),
            pl.BlockSpec((_R, 1), lambda i, j: (i, 0)),
        ],
        out_specs=[
            pl.BlockSpec((_R, _C), lambda i, j: (i, 0)),
            pl.BlockSpec((_R, 1), lambda i, j: (i, 0)),
            pl.BlockSpec((_R, 1), lambda i, j: (i, 0)),
            pl.BlockSpec((_R, 1), lambda i, j: (i, 0)),
        ],
        out_shape=[
            jax.ShapeDtypeStruct((n, _C), jnp.float32),
            jax.ShapeDtypeStruct((n, 1), jnp.int32),
            jax.ShapeDtypeStruct((n, 1), jnp.int32),
            jax.ShapeDtypeStruct((n, 1), jnp.float32),
        ],
        scratch_shapes=[
            pltpu.VMEM((_R, _C), jnp.float32),
            pltpu.VMEM((_R, _C), jnp.bfloat16),
            pltpu.VMEM((_R, 1), jnp.float32),
            pltpu.VMEM((_R, 1), jnp.float32),
            pltpu.VMEM((_R, 1), jnp.int32),
        ],
        compiler_params=pltpu.CompilerParams(
            dimension_semantics=("parallel", "arbitrary")),
    )(xf, wpre_bf, bpre, cbt_bf, csq, noise)


def _out_call(lat, q_noisy, yf, wpost_bf, bpost, mind):
    n = lat.shape[0]
    nr = n // _R
    return pl.pallas_call(
        _out_body,
        grid=(nr,),
        in_specs=[
            pl.BlockSpec((_R, _C), lambda i: (i, 0)),
            pl.BlockSpec((_R, _C), lambda i: (i, 0)),
            pl.BlockSpec((_R, _C), lambda i: (i, 0)),
            pl.BlockSpec((_C, _C), lambda i: (0, 0)),
            pl.BlockSpec((1, _C), lambda i: (0, 0)),
            pl.BlockSpec((_R, 1), lambda i: (i, 0)),
        ],
        out_specs=[
            pl.BlockSpec((_R, _C), lambda i: (i, 0)),
            pl.BlockSpec((1, 1, 128), lambda i: (i, 0, 0)),
            pl.BlockSpec((1, 1, 128), lambda i: (i, 0, 0)),
            pl.BlockSpec((1, 1, 128), lambda i: (i, 0, 0)),
        ],
        out_shape=[
            jax.ShapeDtypeStruct((n, _C), jnp.float32),
            jax.ShapeDtypeStruct((nr, 1, 128), jnp.float32),
            jax.ShapeDtypeStruct((nr, 1, 128), jnp.float32),
            jax.ShapeDtypeStruct((nr, 1, 128), jnp.float32),
        ],
        compiler_params=pltpu.CompilerParams(
            dimension_semantics=("parallel",)),
    )(lat, q_noisy, yf, wpost_bf, bpost, mind)


_NCHUNK = 1


def kernel(x, y, noise, W_pre, b_pre, W_post, b_post, codebook):
    xf = x.reshape(_N, _C)
    yf = y.reshape(_N, _C)
    wpre_bf = W_pre.astype(jnp.bfloat16)
    wpost_bf = W_post.astype(jnp.bfloat16)
    bpre = b_pre.reshape(1, _C)
    bpost = b_post.reshape(1, _C)
    cbt_bf = codebook.T.astype(jnp.bfloat16)

    csq_col = pl.pallas_call(
        _csq_body,
        out_shape=jax.ShapeDtypeStruct((_O, 1), jnp.float32),
    )(codebook)
    csq = csq_col.reshape(1, _O)

    nc = _N // _NCHUNK
    dist_res = []
    gathers = []
    for c in range(_NCHUNK):
        s = slice(c * nc, (c + 1) * nc)
        dist_res.append(_dist_call(xf[s], noise[s], wpre_bf, bpre,
                                   cbt_bf, csq))
        gathers.append(_sc_gather(codebook, dist_res[c][2].reshape(1, nc)))
    outs = []
    for c in range(_NCHUNK):
        lat, _idet, _inoisy, mind = dist_res[c]
        s = slice(c * nc, (c + 1) * nc)
        outs.append(_out_call(lat, gathers[c], yf[s], wpost_bf, bpost, mind))

    out_f = jnp.concatenate([o[0] for o in outs], axis=0)
    rp = jnp.concatenate([o[1] for o in outs], axis=0)
    cp = jnp.concatenate([o[2] for o in outs], axis=0)
    ep = jnp.concatenate([o[3] for o in outs], axis=0)

    lossv = pl.pallas_call(
        _loss_body,
        out_shape=jax.ShapeDtypeStruct((1, 1, 128), jnp.float32),
    )(rp, cp, ep)

    return out_f.reshape(_B, _E, _C), lossv[0, 0, 0]


# speedup vs baseline: 1.2127x; 1.0009x over previous
import jax
import jax.numpy as jnp
from jax.experimental import pallas as pl
from jax.experimental.pallas import tpu as pltpu
from jax.experimental.pallas import tpu_sc as plsc

_B, _E, _C, _O = 16, 576, 256, 8192
_N = _B * _E
_BETA = 0.25
_NOISE_STD = 0.5
_R = 512
_NR = _N // _R
_KT = 8192
_NK = _O // _KT
_GW = 128


def _csq_body(cb_ref, csq_ref):
    c = cb_ref[...]
    csq_ref[...] = jnp.sum(c * c, axis=1, keepdims=True)


def _dist_body(x_ref, wpre_ref, bpre_ref, cbt_ref, csq_ref, noise_ref,
               lat_ref, idet_ref, inoisy_ref, mind_ref,
               lat_s, lath_s, zsq_s, rmin_s, ridx_s):
    j = pl.program_id(1)

    @pl.when(j == 0)
    def _init():
        lat0 = jnp.dot(x_ref[...].astype(jnp.bfloat16), wpre_ref[...],
                       preferred_element_type=jnp.float32) + bpre_ref[...]
        lat_s[...] = lat0
        lath_s[...] = lat0.astype(jnp.bfloat16)
        lat_ref[...] = lat0
        zsq_s[...] = jnp.sum(lat0 * lat0, axis=1, keepdims=True)
        rmin_s[...] = jnp.full((_R, 1), jnp.inf, jnp.float32)
        ridx_s[...] = jnp.zeros((_R, 1), jnp.int32)

    mm = jnp.dot(lath_s[...], cbt_ref[...],
                 preferred_element_type=jnp.float32)
    dist = (zsq_s[...] + csq_ref[...]) - 2.0 * mm
    tmin = jnp.min(dist, axis=1, keepdims=True)
    gidx = jax.lax.broadcasted_iota(jnp.int32, (_R, _KT), 1)
    tidx = (jnp.min(jnp.where(dist == tmin, gidx, jnp.int32(2 ** 30)),
                    axis=1, keepdims=True) + j * _KT)
    upd = tmin < rmin_s[...]
    rmin_s[...] = jnp.where(upd, tmin, rmin_s[...])
    ridx_s[...] = jnp.where(upd, tidx, ridx_s[...])

    @pl.when(j == _NK - 1)
    def _fin():
        idet = ridx_s[...]
        idet_ref[...] = idet
        mind_ref[...] = rmin_s[...]
        off = jnp.round(noise_ref[...] * _NOISE_STD).astype(jnp.int32)
        inoisy_ref[...] = jnp.clip(idet + off, 0, _O - 1)


def _sc_gather(codebook, idx_flat):
    n = idx_flat.shape[1]
    per = n // 32
    nstream = 4
    sub = per // nstream
    idx3 = idx_flat.reshape(32, nstream, sub)
    mesh = plsc.VectorSubcoreMesh(core_axis_name="core",
                                  subcore_axis_name="subcore")

    @pl.kernel(out_type=jax.ShapeDtypeStruct((n, _C), jnp.float32),
               mesh=mesh,
               scratch_types=[pltpu.VMEM((nstream, sub), jnp.int32),
                              pltpu.VMEM((per, _C), jnp.float32)]
               + [pltpu.SemaphoreType.DMA] * nstream)
    def kern(cb_hbm, i_hbm, o_hbm, ivmem, ovmem, *sems):
        c = jax.lax.axis_index("core")
        s = jax.lax.axis_index("subcore")
        u = c * 16 + s
        pltpu.sync_copy(i_hbm.at[u], ivmem)
        cps = [pltpu.make_async_copy(cb_hbm.at[ivmem.at[k]],
                                     ovmem.at[pl.ds(k * sub, sub)],
                                     sems[k])
               for k in range(nstream)]
        for cp in cps:
            cp.start()
        for cp in cps:
            cp.wait()
        pltpu.sync_copy(ovmem, o_hbm.at[pl.ds(u * per, per)])

    return kern(codebook, idx3)


def _out_body(lat_ref, q_ref, y_ref, wpost_ref, bpost_ref, mind_ref,
              out_ref, rp_ref, cp_ref, ep_ref):
    lat = lat_ref[...]
    q = q_ref[...]
    st = lat + (q - lat)
    o = jnp.dot(st.astype(jnp.bfloat16), wpost_ref[...],
                preferred_element_type=jnp.float32) + bpost_ref[...]
    out_ref[...] = o
    d0 = o - y_ref[...]
    d2 = q - lat
    z128 = jnp.zeros((1, 1, 128), jnp.float32)
    rsum = jnp.sum(jnp.sum(d0 * d0, axis=1, keepdims=True), axis=0,
                   keepdims=True)
    esum = jnp.sum(jnp.sum(d2 * d2, axis=1, keepdims=True), axis=0,
                   keepdims=True)
    csum = jnp.sum(mind_ref[...], axis=0, keepdims=True)
    rp_ref[...] = z128 + rsum.reshape(1, 1, 1)
    ep_ref[...] = z128 + esum.reshape(1, 1, 1)
    cp_ref[...] = z128 + csum.reshape(1, 1, 1)


def _loss_body(rp_ref, cp_ref, ep_ref, loss_ref):
    ntot = float(_N * _C)
    recon = jnp.sum(rp_ref[...], axis=0, keepdims=True) / ntot
    commit = _BETA * (jnp.sum(cp_ref[...], axis=0, keepdims=True) / ntot)
    embed = jnp.sum(ep_ref[...], axis=0, keepdims=True) / ntot
    loss_ref[...] = (recon + commit) + embed


def _dist_call(xf, noise, wpre_bf, bpre, cbt_bf, csq):
    n = xf.shape[0]
    nr = n // _R
    return pl.pallas_call(
        _dist_body,
        grid=(nr, _NK),
        in_specs=[
            pl.BlockSpec((_R, _C), lambda i, j: (i, 0)),
            pl.BlockSpec((_C, _C), lambda i, j: (0, 0)),
            pl.BlockSpec((1, _C), lambda i, j: (0, 0)),
            pl.BlockSpec((_C, _KT), lambda i, j: (0, j)),
            pl.BlockSpec((1, _KT), lambda i, j: (0, j)),
            pl.BlockSpec((_R, 1), lambda i, j: (i, 0)),
        ],
        out_specs=[
            pl.BlockSpec((_R, _C), lambda i, j: (i, 0)),
            pl.BlockSpec((_R, 1), lambda i, j: (i, 0)),
            pl.BlockSpec((_R, 1), lambda i, j: (i, 0)),
            pl.BlockSpec((_R, 1), lambda i, j: (i, 0)),
        ],
        out_shape=[
            jax.ShapeDtypeStruct((n, _C), jnp.float32),
            jax.ShapeDtypeStruct((n, 1), jnp.int32),
            jax.ShapeDtypeStruct((n, 1), jnp.int32),
            jax.ShapeDtypeStruct((n, 1), jnp.float32),
        ],
        scratch_shapes=[
            pltpu.VMEM((_R, _C), jnp.float32),
            pltpu.VMEM((_R, _C), jnp.bfloat16),
            pltpu.VMEM((_R, 1), jnp.float32),
            pltpu.VMEM((_R, 1), jnp.float32),
            pltpu.VMEM((_R, 1), jnp.int32),
        ],
        compiler_params=pltpu.CompilerParams(
            dimension_semantics=("parallel", "arbitrary")),
    )(xf, wpre_bf, bpre, cbt_bf, csq, noise)


def _out_call(lat, q_noisy, yf, wpost_bf, bpost, mind):
    n = lat.shape[0]
    nr = n // _R
    return pl.pallas_call(
        _out_body,
        grid=(nr,),
        in_specs=[
            pl.BlockSpec((_R, _C), lambda i: (i, 0)),
            pl.BlockSpec((_R, _C), lambda i: (i, 0)),
            pl.BlockSpec((_R, _C), lambda i: (i, 0)),
            pl.BlockSpec((_C, _C), lambda i: (0, 0)),
            pl.BlockSpec((1, _C), lambda i: (0, 0)),
            pl.BlockSpec((_R, 1), lambda i: (i, 0)),
        ],
        out_specs=[
            pl.BlockSpec((_R, _C), lambda i: (i, 0)),
            pl.BlockSpec((1, 1, 128), lambda i: (i, 0, 0)),
            pl.BlockSpec((1, 1, 128), lambda i: (i, 0, 0)),
            pl.BlockSpec((1, 1, 128), lambda i: (i, 0, 0)),
        ],
        out_shape=[
            jax.ShapeDtypeStruct((n, _C), jnp.float32),
            jax.ShapeDtypeStruct((nr, 1, 128), jnp.float32),
            jax.ShapeDtypeStruct((nr, 1, 128), jnp.float32),
            jax.ShapeDtypeStruct((nr, 1, 128), jnp.float32),
        ],
        compiler_params=pltpu.CompilerParams(
            dimension_semantics=("parallel",)),
    )(lat, q_noisy, yf, wpost_bf, bpost, mind)


_NCHUNK = 1


def kernel(x, y, noise, W_pre, b_pre, W_post, b_post, codebook):
    xf = x.reshape(_N, _C)
    yf = y.reshape(_N, _C)
    wpre_bf = W_pre.astype(jnp.bfloat16)
    wpost_bf = W_post.astype(jnp.bfloat16)
    bpre = b_pre.reshape(1, _C)
    bpost = b_post.reshape(1, _C)
    cbt_bf = codebook.T.astype(jnp.bfloat16)

    csq_col = pl.pallas_call(
        _csq_body,
        out_shape=jax.ShapeDtypeStruct((_O, 1), jnp.float32),
    )(codebook)
    csq = csq_col.reshape(1, _O)

    nc = _N // _NCHUNK
    dist_res = []
    gathers = []
    for c in range(_NCHUNK):
        s = slice(c * nc, (c + 1) * nc)
        dist_res.append(_dist_call(xf[s], noise[s], wpre_bf, bpre,
                                   cbt_bf, csq))
        gathers.append(_sc_gather(codebook, dist_res[c][2].reshape(1, nc)))
    outs = []
    for c in range(_NCHUNK):
        lat, _idet, _inoisy, mind = dist_res[c]
        s = slice(c * nc, (c + 1) * nc)
        outs.append(_out_call(lat, gathers[c], yf[s], wpost_bf, bpost, mind))

    out_f = jnp.concatenate([o[0] for o in outs], axis=0)
    rp = jnp.concatenate([o[1] for o in outs], axis=0)
    cp = jnp.concatenate([o[2] for o in outs], axis=0)
    ep = jnp.concatenate([o[3] for o in outs], axis=0)

    lossv = pl.pallas_call(
        _loss_body,
        out_shape=jax.ShapeDtypeStruct((1, 1, 128), jnp.float32),
    )(rp, cp, ep)

    return out_f.reshape(_B, _E, _C), lossv[0, 0, 0]
